# Initial kernel scaffold; baseline (speedup 1.0000x reference)
import functools

import jax
import jax.numpy as jnp
from jax import lax
from jax.experimental import pallas as pl
from jax.experimental.pallas import tpu as pltpu
from jax.experimental.pallas import tpu_sc as plsc

N_NODES = 10000
N_EDGES = 320000
D_IN = 128
D_EDGE = 16
D = 64

NC = 2
NS = 16
NW = NC * NS
EPW = N_EDGES // NW
EB = 80
NBLK = EPW // EB
ACC_W = 80
RPT = N_NODES // NS
RCH = 125
NCH = RPT // RCH


def _proj_body(x_ref, wq_ref, bq_ref, wk_ref, bk_ref, wv_ref, bv_ref,
               q_ref, k_ref, v_ref):
    xb = x_ref[...]
    q_ref[...] = (jnp.dot(xb, wq_ref[...], preferred_element_type=jnp.float32)
                  + bq_ref[...]) * 0.125
    k_ref[...] = jnp.dot(xb, wk_ref[...], preferred_element_type=jnp.float32) + bk_ref[...]
    v_ref[...] = jnp.dot(xb, wv_ref[...], preferred_element_type=jnp.float32) + bv_ref[...]


def _proj(x, wq, bq, wk, bk, wv, bv):
    n, din = x.shape
    rb = 2000
    w_spec = pl.BlockSpec((din, D), lambda i: (0, 0))
    b_spec = pl.BlockSpec((1, D), lambda i: (0, 0))
    return pl.pallas_call(
        _proj_body,
        grid=(n // rb,),
        in_specs=[pl.BlockSpec((rb, din), lambda i: (i, 0)),
                  w_spec, b_spec, w_spec, b_spec, w_spec, b_spec],
        out_specs=[pl.BlockSpec((rb, D), lambda i: (i, 0))] * 3,
        out_shape=[jax.ShapeDtypeStruct((n, D), jnp.float32)] * 3,
    )(x, wq, bq.reshape(1, D), wk, bk.reshape(1, D), wv, bv.reshape(1, D))


def _edge_body(a_ref, w0_ref, b0_ref, w1_ref, b1_ref, e0_ref, e1_ref):
    ab = a_ref[...]
    e0_ref[...] = jnp.dot(ab, w0_ref[...], preferred_element_type=jnp.float32) + b0_ref[...]
    e1_ref[...] = jnp.dot(ab, w1_ref[...], preferred_element_type=jnp.float32) + b1_ref[...]


def _edge_embed(edge_attr, w0, b0, w1, b1):
    rb = 8000
    w_spec = pl.BlockSpec((D_EDGE, D), lambda i: (0, 0))
    b_spec = pl.BlockSpec((1, D), lambda i: (0, 0))
    return pl.pallas_call(
        _edge_body,
        grid=(N_EDGES // rb,),
        in_specs=[pl.BlockSpec((rb, D_EDGE), lambda i: (i, 0)),
                  w_spec, b_spec, w_spec, b_spec],
        out_specs=[pl.BlockSpec((rb, D), lambda i: (i, 0))] * 2,
        out_shape=[jax.ShapeDtypeStruct((N_EDGES, D), jnp.float32)] * 2,
    )(edge_attr, w0, b0.reshape(1, D), w1, b1.reshape(1, D))


def _mid_body(p0_ref, p1_ref, wq_ref, bq_ref, wk_ref, bk_ref, wv_ref, bv_ref,
              q_ref, k_ref, v_ref):
    ps = p0_ref[...] + p1_ref[...]
    num = ps[:, :D]
    den = ps[:, D:D + 1]
    h = jnp.maximum(num / (den + 1e-16), 0.0)
    q_ref[...] = (jnp.dot(h, wq_ref[...], preferred_element_type=jnp.float32)
                  + bq_ref[...]) * 0.125
    k_ref[...] = jnp.dot(h, wk_ref[...], preferred_element_type=jnp.float32) + bk_ref[...]
    v_ref[...] = jnp.dot(h, wv_ref[...], preferred_element_type=jnp.float32) + bv_ref[...]


def _mid(p, wq, bq, wk, bk, wv, bv):
    rb = 2000
    nb = N_NODES // rb
    w_spec = pl.BlockSpec((D, D), lambda i: (0, 0))
    b_spec = pl.BlockSpec((1, D), lambda i: (0, 0))
    return pl.pallas_call(
        _mid_body,
        grid=(nb,),
        in_specs=[pl.BlockSpec((rb, ACC_W), lambda i: (i, 0)),
                  pl.BlockSpec((rb, ACC_W), lambda i: (i + nb, 0)),
                  w_spec, b_spec, w_spec, b_spec, w_spec, b_spec],
        out_specs=[pl.BlockSpec((rb, D), lambda i: (i, 0))] * 3,
        out_shape=[jax.ShapeDtypeStruct((N_NODES, D), jnp.float32)] * 3,
    )(p, p, wq, bq.reshape(1, D), wk, bk.reshape(1, D), wv, bv.reshape(1, D))


def _final_body(p0_ref, p1_ref, o_ref):
    ps = p0_ref[...] + p1_ref[...]
    o_ref[...] = ps[:, :D] / (ps[:, D:D + 1] + 1e-16)


def _final(p):
    rb = 2000
    nb = N_NODES // rb
    return pl.pallas_call(
        _final_body,
        grid=(nb,),
        in_specs=[pl.BlockSpec((rb, ACC_W), lambda i: (i, 0)),
                  pl.BlockSpec((rb, ACC_W), lambda i: (i + nb, 0))],
        out_specs=pl.BlockSpec((rb, D), lambda i: (i, 0)),
        out_shape=jax.ShapeDtypeStruct((N_NODES, D), jnp.float32),
    )(p, p)


def _sc_attention(q, k, v, e, src, dst):
    mesh = plsc.VectorSubcoreMesh(core_axis_name="c", subcore_axis_name="s")

    @functools.partial(
        pl.kernel,
        out_type=jax.ShapeDtypeStruct((NC * N_NODES, ACC_W), jnp.float32),
        mesh=mesh,
        scratch_types=[
            pltpu.VMEM((EB,), jnp.int32),
            pltpu.VMEM((EB,), jnp.int32),
            pltpu.VMEM((EB, D), jnp.float32),
            pltpu.VMEM((EB, D), jnp.float32),
            pltpu.VMEM((EB, D), jnp.float32),
            pltpu.VMEM((EB, D), jnp.float32),
            pltpu.VMEM((EB, ACC_W), jnp.float32),
            pltpu.VMEM((RCH, ACC_W), jnp.float32),
            pltpu.VMEM_SHARED((N_NODES, ACC_W), jnp.float32),
            pltpu.SemaphoreType.DMA,
            pltpu.SemaphoreType.DMA,
            pltpu.SemaphoreType.DMA,
            pltpu.SemaphoreType.DMA,
        ],
    )
    def sc_kernel(q_hbm, k_hbm, v_hbm, e_hbm, src_hbm, dst_hbm, out_hbm,
                  src_v, dst_v, qg, kg, vg, eg, sbuf, zbuf, acc,
                  sem_q, sem_k, sem_v, sem_e):
        c = lax.axis_index("c")
        s = lax.axis_index("s")
        wid = c * NS + s

        @pl.loop(0, RCH)
        def _zero_rows(i):
            for j in range(ACC_W // 16):
                zbuf[i, pl.ds(j * 16, 16)] = jnp.zeros((16,), jnp.float32)

        @pl.loop(0, NCH)
        def _zero_acc(jc):
            off = s * RPT + jc * RCH
            pltpu.sync_copy(zbuf, acc.at[pl.ds(off, RCH)])

        plsc.subcore_barrier()

        lane = lax.iota(jnp.int32, (16,), 0)
        onehot0 = jnp.where(lane == 0, 1.0, 0.0).astype(jnp.float32)
        base0 = wid * EPW

        @pl.loop(0, NBLK)
        def _blocks(b):
            base = base0 + b * EB
            pltpu.sync_copy(src_hbm.at[pl.ds(base, EB)], src_v)
            pltpu.sync_copy(dst_hbm.at[pl.ds(base, EB)], dst_v)
            cq = pltpu.async_copy(q_hbm.at[dst_v], qg, sem_q)
            ck = pltpu.async_copy(k_hbm.at[src_v], kg, sem_k)
            cv = pltpu.async_copy(v_hbm.at[src_v], vg, sem_v)
            ce = pltpu.async_copy(e_hbm.at[pl.ds(base, EB)], eg, sem_e)
            cq.wait()
            ck.wait()
            cv.wait()
            ce.wait()

            @pl.loop(0, EB)
            def _edges(i):
                e0 = eg[i, pl.ds(0, 16)]
                e1 = eg[i, pl.ds(16, 16)]
                e2 = eg[i, pl.ds(32, 16)]
                e3 = eg[i, pl.ds(48, 16)]
                prod = (qg[i, pl.ds(0, 16)] * (kg[i, pl.ds(0, 16)] + e0)
                        + qg[i, pl.ds(16, 16)] * (kg[i, pl.ds(16, 16)] + e1)
                        + qg[i, pl.ds(32, 16)] * (kg[i, pl.ds(32, 16)] + e2)
                        + qg[i, pl.ds(48, 16)] * (kg[i, pl.ds(48, 16)] + e3))
                a = jnp.sum(prod)
                ew = jnp.exp(jnp.full((16,), a, jnp.float32))
                sbuf[i, pl.ds(0, 16)] = ew * (vg[i, pl.ds(0, 16)] + e0)
                sbuf[i, pl.ds(16, 16)] = ew * (vg[i, pl.ds(16, 16)] + e1)
                sbuf[i, pl.ds(32, 16)] = ew * (vg[i, pl.ds(32, 16)] + e2)
                sbuf[i, pl.ds(48, 16)] = ew * (vg[i, pl.ds(48, 16)] + e3)
                sbuf[i, pl.ds(D, 16)] = ew * onehot0

            pltpu.sync_copy(sbuf, acc.at[dst_v], add=True)

        plsc.subcore_barrier()

        @pl.loop(0, NCH)
        def _copy_out(jc):
            off = s * RPT + jc * RCH
            pltpu.sync_copy(acc.at[pl.ds(off, RCH)], zbuf)
            pltpu.sync_copy(zbuf, out_hbm.at[pl.ds(c * N_NODES + off, RCH)])

    return sc_kernel(q, k, v, e, src, dst)


def kernel(x, edge_index, edge_attr, Wq0, bq0, Wk0, bk0, Wv0, bv0, We0, be0,
           Wq1, bq1, Wk1, bk1, Wv1, bv1, We1, be1):
    src = edge_index[0]
    dst = edge_index[1]
    q0, k0, v0 = _proj(x, Wq0, bq0, Wk0, bk0, Wv0, bv0)
    e0, e1 = _edge_embed(edge_attr, We0, be0, We1, be1)
    p0 = _sc_attention(q0, k0, v0, e0, src, dst)
    q1, k1, v1 = _mid(p0, Wq1, bq1, Wk1, bk1, Wv1, bv1)
    p1 = _sc_attention(q1, k1, v1, e1, src, dst)
    return _final(p1)


# speedup vs baseline: 7.7714x; 7.7714x over previous
import functools

import jax
import jax.numpy as jnp
from jax import lax
from jax.experimental import pallas as pl
from jax.experimental.pallas import tpu as pltpu
from jax.experimental.pallas import tpu_sc as plsc

N_NODES = 10000
N_EDGES = 320000
D_IN = 128
D_EDGE = 16
D = 64

NC = 2
NS = 16
NW = NC * NS
EPW = N_EDGES // NW
EB = 80
NBLK = EPW // EB
ACC_W = 80
RCH = 80
NCHT = N_NODES // RCH
NCH_LOOP = -(-NCHT // NS)


def _proj_body(x_ref, wq_ref, bq_ref, wk_ref, bk_ref, wv_ref, bv_ref,
               q_ref, kv_ref):
    xb = x_ref[...]
    qb = (jnp.dot(xb, wq_ref[...], preferred_element_type=jnp.float32)
          + bq_ref[...]) * 0.125
    q_ref[...] = jnp.concatenate([qb, jnp.zeros_like(qb)], axis=1)
    kb = jnp.dot(xb, wk_ref[...], preferred_element_type=jnp.float32) + bk_ref[...]
    vb = jnp.dot(xb, wv_ref[...], preferred_element_type=jnp.float32) + bv_ref[...]
    kv_ref[...] = jnp.concatenate([kb, vb], axis=1)


def _proj(x, wq, bq, wk, bk, wv, bv):
    n, din = x.shape
    rb = 2000
    w_spec = pl.BlockSpec((din, D), lambda i: (0, 0))
    b_spec = pl.BlockSpec((1, D), lambda i: (0, 0))
    return pl.pallas_call(
        _proj_body,
        grid=(n // rb,),
        in_specs=[pl.BlockSpec((rb, din), lambda i: (i, 0)),
                  w_spec, b_spec, w_spec, b_spec, w_spec, b_spec],
        out_specs=[pl.BlockSpec((rb, 2 * D), lambda i: (i, 0))] * 2,
        out_shape=[jax.ShapeDtypeStruct((n, 2 * D), jnp.float32)] * 2,
    )(x, wq, bq.reshape(1, D), wk, bk.reshape(1, D), wv, bv.reshape(1, D))


def _edge_body(a_ref, w0_ref, b0_ref, w1_ref, b1_ref, e0_ref, e1_ref):
    ab = a_ref[...]
    e0_ref[...] = jnp.dot(ab, w0_ref[...], preferred_element_type=jnp.float32) + b0_ref[...]
    e1_ref[...] = jnp.dot(ab, w1_ref[...], preferred_element_type=jnp.float32) + b1_ref[...]


def _edge_embed(edge_attr, w0, b0, w1, b1):
    rb = 8000
    w_spec = pl.BlockSpec((D_EDGE, D), lambda i: (0, 0))
    b_spec = pl.BlockSpec((1, D), lambda i: (0, 0))
    return pl.pallas_call(
        _edge_body,
        grid=(N_EDGES // rb,),
        in_specs=[pl.BlockSpec((rb, D_EDGE), lambda i: (i, 0)),
                  w_spec, b_spec, w_spec, b_spec],
        out_specs=[pl.BlockSpec((rb, D), lambda i: (i, 0))] * 2,
        out_shape=[jax.ShapeDtypeStruct((N_EDGES, D), jnp.float32)] * 2,
    )(edge_attr, w0, b0.reshape(1, D), w1, b1.reshape(1, D))


def _mid_body(p0_ref, p1_ref, wq_ref, bq_ref, wk_ref, bk_ref, wv_ref, bv_ref,
              q_ref, kv_ref):
    ps = p0_ref[...] + p1_ref[...]
    num = ps[:, :D]
    den = ps[:, D:D + 1]
    h = jnp.maximum(num / (den + 1e-16), 0.0)
    qb = (jnp.dot(h, wq_ref[...], preferred_element_type=jnp.float32)
          + bq_ref[...]) * 0.125
    q_ref[...] = jnp.concatenate([qb, jnp.zeros_like(qb)], axis=1)
    kb = jnp.dot(h, wk_ref[...], preferred_element_type=jnp.float32) + bk_ref[...]
    vb = jnp.dot(h, wv_ref[...], preferred_element_type=jnp.float32) + bv_ref[...]
    kv_ref[...] = jnp.concatenate([kb, vb], axis=1)


def _mid(p, wq, bq, wk, bk, wv, bv):
    rb = 2000
    nb = N_NODES // rb
    w_spec = pl.BlockSpec((D, D), lambda i: (0, 0))
    b_spec = pl.BlockSpec((1, D), lambda i: (0, 0))
    return pl.pallas_call(
        _mid_body,
        grid=(nb,),
        in_specs=[pl.BlockSpec((rb, ACC_W), lambda i: (i, 0)),
                  pl.BlockSpec((rb, ACC_W), lambda i: (i + nb, 0)),
                  w_spec, b_spec, w_spec, b_spec, w_spec, b_spec],
        out_specs=[pl.BlockSpec((rb, 2 * D), lambda i: (i, 0))] * 2,
        out_shape=[jax.ShapeDtypeStruct((N_NODES, 2 * D), jnp.float32)] * 2,
    )(p, p, wq, bq.reshape(1, D), wk, bk.reshape(1, D), wv, bv.reshape(1, D))


def _final_body(p0_ref, p1_ref, o_ref):
    ps = p0_ref[...] + p1_ref[...]
    o_ref[...] = ps[:, :D] / (ps[:, D:D + 1] + 1e-16)


def _final(p):
    rb = 2000
    nb = N_NODES // rb
    return pl.pallas_call(
        _final_body,
        grid=(nb,),
        in_specs=[pl.BlockSpec((rb, ACC_W), lambda i: (i, 0)),
                  pl.BlockSpec((rb, ACC_W), lambda i: (i + nb, 0))],
        out_specs=pl.BlockSpec((rb, D), lambda i: (i, 0)),
        out_shape=jax.ShapeDtypeStruct((N_NODES, D), jnp.float32),
    )(p, p)


_GATHER_DNUMS = lax.GatherDimensionNumbers(
    offset_dims=(), collapsed_slice_dims=(0,), start_index_map=(0,))


def _lane_shuffle(vv, idx):
    return lax.gather(vv, idx[:, None], _GATHER_DNUMS, (1,),
                      mode=lax.GatherScatterMode.PROMISE_IN_BOUNDS)

def _sc_attention(q, kv, e2, src3, dst3):
    mesh = plsc.VectorSubcoreMesh(core_axis_name="c", subcore_axis_name="s")

    @functools.partial(
        pl.kernel,
        out_type=jax.ShapeDtypeStruct((NC * N_NODES, ACC_W), jnp.float32),
        mesh=mesh,
        scratch_types=[
            pltpu.VMEM((NBLK, EB), jnp.int32),
            pltpu.VMEM((NBLK, EB), jnp.int32),
            pltpu.VMEM((EB, 2 * D), jnp.float32),
            pltpu.VMEM((EB, 2 * D), jnp.float32),
            pltpu.VMEM((EB // 2, 2 * D), jnp.float32),
            pltpu.VMEM((EB, ACC_W), jnp.float32),
            pltpu.VMEM((RCH, ACC_W), jnp.float32),
            pltpu.VMEM_SHARED((N_NODES, ACC_W), jnp.float32),
            pltpu.SemaphoreType.DMA,
            pltpu.SemaphoreType.DMA,
            pltpu.SemaphoreType.DMA,
        ],
        compiler_params=pltpu.CompilerParams(use_tc_tiling_on_sc=False),
    )
    def sc_kernel(q_hbm, kv_hbm, e_hbm, src_hbm, dst_hbm, out_hbm,
                  src2d, dst2d, qg, kvg, eg, sbuf, zbuf, acc,
                  sem_q, sem_kv, sem_e):
        c = lax.axis_index("c")
        s = lax.axis_index("s")
        wid = c * NS + s

        @pl.loop(0, RCH)
        def _zero_rows(i):
            for j in range(ACC_W // 16):
                zbuf[i, pl.ds(j * 16, 16)] = jnp.zeros((16,), jnp.float32)

        @pl.loop(0, NCH_LOOP)
        def _zero_acc(jc):
            m = s + jc * NS
            @pl.when(m < NCHT)
            def _():
                pltpu.sync_copy(zbuf, acc.at[pl.ds(m * RCH, RCH)])

        pltpu.sync_copy(src_hbm.at[wid], src2d)
        pltpu.sync_copy(dst_hbm.at[wid], dst2d)

        plsc.subcore_barrier()

        lane = lax.iota(jnp.int32, 16)
        onehot0 = jnp.where(lane == 0, 1.0, 0.0).astype(jnp.float32)
        ebase0 = wid * (EPW // 2)

        @pl.loop(0, NBLK)
        def _blocks(b):
            cq = pltpu.async_copy(q_hbm.at[dst2d.at[b]], qg, sem_q)
            ckv = pltpu.async_copy(kv_hbm.at[src2d.at[b]], kvg, sem_kv)
            ce = pltpu.async_copy(
                e_hbm.at[pl.ds(ebase0 + b * (EB // 2), EB // 2)], eg, sem_e)
            cq.wait()
            ckv.wait()
            ce.wait()

            @pl.loop(0, EB // 2)
            def _edge_pairs(j):
                for half in range(2):
                    i = 2 * j + half
                    eoff = half * D
                    ev = [eg[j, pl.ds(eoff + 16 * t, 16)] for t in range(4)]
                    prod = (qg[i, pl.ds(0, 16)] * (kvg[i, pl.ds(0, 16)] + ev[0])
                            + qg[i, pl.ds(16, 16)] * (kvg[i, pl.ds(16, 16)] + ev[1])
                            + qg[i, pl.ds(32, 16)] * (kvg[i, pl.ds(32, 16)] + ev[2])
                            + qg[i, pl.ds(48, 16)] * (kvg[i, pl.ds(48, 16)] + ev[3]))
                    for sh in (8, 4, 2, 1):
                        idx = lane ^ sh
                        prod = prod + _lane_shuffle(prod, idx)
                    ew = jnp.exp(prod)
                    for t in range(4):
                        sbuf[i, pl.ds(16 * t, 16)] = ew * (
                            kvg[i, pl.ds(D + 16 * t, 16)] + ev[t])
                    sbuf[i, pl.ds(D, 16)] = ew * onehot0

            pltpu.sync_copy(sbuf, acc.at[dst2d.at[b]], add=True)

        plsc.subcore_barrier()

        @pl.loop(0, NCH_LOOP)
        def _copy_out(jc):
            m = s + jc * NS
            @pl.when(m < NCHT)
            def _():
                off = m * RCH
                pltpu.sync_copy(acc.at[pl.ds(off, RCH)], zbuf)
                pltpu.sync_copy(zbuf, out_hbm.at[pl.ds(c * N_NODES + off, RCH)])

    return sc_kernel(q, kv, e2, src3, dst3)


def kernel(x, edge_index, edge_attr, Wq0, bq0, Wk0, bk0, Wv0, bv0, We0, be0,
           Wq1, bq1, Wk1, bk1, Wv1, bv1, We1, be1):
    src3 = edge_index[0].reshape(NW, NBLK, EB)
    dst3 = edge_index[1].reshape(NW, NBLK, EB)
    q0, kv0 = _proj(x, Wq0, bq0, Wk0, bk0, Wv0, bv0)
    e0, e1 = _edge_embed(edge_attr, We0, be0, We1, be1)
    p0 = _sc_attention(q0, kv0, e0.reshape(N_EDGES // 2, 2 * D), src3, dst3)
    q1, kv1 = _mid(p0, Wq1, bq1, Wk1, bk1, Wv1, bv1)
    p1 = _sc_attention(q1, kv1, e1.reshape(N_EDGES // 2, 2 * D), src3, dst3)
    return _final(p1)


# speedup vs baseline: 7.9232x; 1.0195x over previous
import functools

import jax
import jax.numpy as jnp
from jax import lax
from jax.experimental import pallas as pl
from jax.experimental.pallas import tpu as pltpu
from jax.experimental.pallas import tpu_sc as plsc

N_NODES = 10000
N_EDGES = 320000
D_IN = 128
D_EDGE = 16
D = 64

NC = 2
NS = 16
NW = NC * NS
EPW = N_EDGES // NW
EB = 80
NBLK = EPW // EB
ACC_W = 80
RCH = 80
NCHT = N_NODES // RCH
NCH_LOOP = -(-NCHT // NS)


def _proj_body(x_ref, wq_ref, bq_ref, wk_ref, bk_ref, wv_ref, bv_ref,
               q_ref, kv_ref):
    xb = x_ref[...]
    q_ref[...] = (jnp.dot(xb, wq_ref[...], preferred_element_type=jnp.float32)
                  + bq_ref[...]) * 0.125
    kb = jnp.dot(xb, wk_ref[...], preferred_element_type=jnp.float32) + bk_ref[...]
    vb = jnp.dot(xb, wv_ref[...], preferred_element_type=jnp.float32) + bv_ref[...]
    kv_ref[...] = jnp.concatenate([kb, vb], axis=1)


def _proj(x, wq, bq, wk, bk, wv, bv):
    n, din = x.shape
    rb = 2000
    w_spec = pl.BlockSpec((din, D), lambda i: (0, 0))
    b_spec = pl.BlockSpec((1, D), lambda i: (0, 0))
    return pl.pallas_call(
        _proj_body,
        grid=(n // rb,),
        in_specs=[pl.BlockSpec((rb, din), lambda i: (i, 0)),
                  w_spec, b_spec, w_spec, b_spec, w_spec, b_spec],
        out_specs=[pl.BlockSpec((rb, D), lambda i: (i, 0)),
                   pl.BlockSpec((rb, 2 * D), lambda i: (i, 0))],
        out_shape=[jax.ShapeDtypeStruct((n, D), jnp.float32),
                   jax.ShapeDtypeStruct((n, 2 * D), jnp.float32)],
    )(x, wq, bq.reshape(1, D), wk, bk.reshape(1, D), wv, bv.reshape(1, D))


def _edge_body(a_ref, w0_ref, b0_ref, w1_ref, b1_ref, e0_ref, e1_ref):
    ab = a_ref[...]
    e0_ref[...] = jnp.dot(ab, w0_ref[...], preferred_element_type=jnp.float32) + b0_ref[...]
    e1_ref[...] = jnp.dot(ab, w1_ref[...], preferred_element_type=jnp.float32) + b1_ref[...]


def _edge_embed(edge_attr, w0, b0, w1, b1):
    rb = 8000
    w_spec = pl.BlockSpec((D_EDGE, D), lambda i: (0, 0))
    b_spec = pl.BlockSpec((1, D), lambda i: (0, 0))
    return pl.pallas_call(
        _edge_body,
        grid=(N_EDGES // rb,),
        in_specs=[pl.BlockSpec((rb, D_EDGE), lambda i: (i, 0)),
                  w_spec, b_spec, w_spec, b_spec],
        out_specs=[pl.BlockSpec((rb, D), lambda i: (i, 0))] * 2,
        out_shape=[jax.ShapeDtypeStruct((N_EDGES, D), jnp.float32)] * 2,
    )(edge_attr, w0, b0.reshape(1, D), w1, b1.reshape(1, D))


def _mid_body(p0_ref, p1_ref, wq_ref, bq_ref, wk_ref, bk_ref, wv_ref, bv_ref,
              q_ref, kv_ref):
    ps = p0_ref[...] + p1_ref[...]
    num = ps[:, :D]
    den = ps[:, D:D + 1]
    h = jnp.maximum(num / (den + 1e-16), 0.0)
    q_ref[...] = (jnp.dot(h, wq_ref[...], preferred_element_type=jnp.float32)
                  + bq_ref[...]) * 0.125
    kb = jnp.dot(h, wk_ref[...], preferred_element_type=jnp.float32) + bk_ref[...]
    vb = jnp.dot(h, wv_ref[...], preferred_element_type=jnp.float32) + bv_ref[...]
    kv_ref[...] = jnp.concatenate([kb, vb], axis=1)


def _mid(p, wq, bq, wk, bk, wv, bv):
    rb = 2000
    nb = N_NODES // rb
    w_spec = pl.BlockSpec((D, D), lambda i: (0, 0))
    b_spec = pl.BlockSpec((1, D), lambda i: (0, 0))
    return pl.pallas_call(
        _mid_body,
        grid=(nb,),
        in_specs=[pl.BlockSpec((rb, ACC_W), lambda i: (i, 0)),
                  pl.BlockSpec((rb, ACC_W), lambda i: (i + nb, 0)),
                  w_spec, b_spec, w_spec, b_spec, w_spec, b_spec],
        out_specs=[pl.BlockSpec((rb, D), lambda i: (i, 0)),
                   pl.BlockSpec((rb, 2 * D), lambda i: (i, 0))],
        out_shape=[jax.ShapeDtypeStruct((N_NODES, D), jnp.float32),
                   jax.ShapeDtypeStruct((N_NODES, 2 * D), jnp.float32)],
    )(p, p, wq, bq.reshape(1, D), wk, bk.reshape(1, D), wv, bv.reshape(1, D))


def _final_body(p0_ref, p1_ref, o_ref):
    ps = p0_ref[...] + p1_ref[...]
    o_ref[...] = ps[:, :D] / (ps[:, D:D + 1] + 1e-16)


def _final(p):
    rb = 2000
    nb = N_NODES // rb
    return pl.pallas_call(
        _final_body,
        grid=(nb,),
        in_specs=[pl.BlockSpec((rb, ACC_W), lambda i: (i, 0)),
                  pl.BlockSpec((rb, ACC_W), lambda i: (i + nb, 0))],
        out_specs=pl.BlockSpec((rb, D), lambda i: (i, 0)),
        out_shape=jax.ShapeDtypeStruct((N_NODES, D), jnp.float32),
    )(p, p)


_GATHER_DNUMS = lax.GatherDimensionNumbers(
    offset_dims=(), collapsed_slice_dims=(0,), start_index_map=(0,))


def _lane_shuffle(vv, idx):
    return lax.gather(vv, idx[:, None], _GATHER_DNUMS, (1,),
                      mode=lax.GatherScatterMode.PROMISE_IN_BOUNDS)

def _sc_attention(q, kv, e2, src3, dst3):
    mesh = plsc.VectorSubcoreMesh(core_axis_name="c", subcore_axis_name="s")

    @functools.partial(
        pl.kernel,
        out_type=jax.ShapeDtypeStruct((NC * N_NODES, ACC_W), jnp.float32),
        mesh=mesh,
        scratch_types=[
            pltpu.VMEM((NBLK, EB), jnp.int32),
            pltpu.VMEM((NBLK, EB), jnp.int32),
            pltpu.VMEM((EB, D), jnp.float32),
            pltpu.VMEM((EB, 2 * D), jnp.float32),
            pltpu.VMEM((EB // 2, 2 * D), jnp.float32),
            pltpu.VMEM((EB, ACC_W), jnp.float32),
            pltpu.VMEM((RCH, ACC_W), jnp.float32),
            pltpu.VMEM_SHARED((N_NODES, ACC_W), jnp.float32),
            pltpu.SemaphoreType.DMA,
            pltpu.SemaphoreType.DMA,
            pltpu.SemaphoreType.DMA,
        ],
        compiler_params=pltpu.CompilerParams(use_tc_tiling_on_sc=False),
    )
    def sc_kernel(q_hbm, kv_hbm, e_hbm, src_hbm, dst_hbm, out_hbm,
                  src2d, dst2d, qg, kvg, eg, sbuf, zbuf, acc,
                  sem_q, sem_kv, sem_e):
        c = lax.axis_index("c")
        s = lax.axis_index("s")
        wid = c * NS + s

        @pl.loop(0, RCH)
        def _zero_rows(i):
            for j in range(ACC_W // 16):
                zbuf[i, pl.ds(j * 16, 16)] = jnp.zeros((16,), jnp.float32)

        @pl.loop(0, NCH_LOOP)
        def _zero_acc(jc):
            m = s + jc * NS
            @pl.when(m < NCHT)
            def _():
                pltpu.sync_copy(zbuf, acc.at[pl.ds(m * RCH, RCH)])

        pltpu.sync_copy(src_hbm.at[wid], src2d)
        pltpu.sync_copy(dst_hbm.at[wid], dst2d)

        plsc.subcore_barrier()

        lane = lax.iota(jnp.int32, 16)
        onehot0 = jnp.where(lane == 0, 1.0, 0.0).astype(jnp.float32)
        ebase0 = wid * (EPW // 2)

        @pl.loop(0, NBLK)
        def _blocks(b):
            cq = pltpu.async_copy(q_hbm.at[dst2d.at[b]], qg, sem_q)
            ckv = pltpu.async_copy(kv_hbm.at[src2d.at[b]], kvg, sem_kv)
            ce = pltpu.async_copy(
                e_hbm.at[pl.ds(ebase0 + b * (EB // 2), EB // 2)], eg, sem_e)
            cq.wait()
            ckv.wait()
            ce.wait()

            @pl.loop(0, EB // 2)
            def _edge_pairs(j):
                for half in range(2):
                    i = 2 * j + half
                    eoff = half * D
                    ev = [eg[j, pl.ds(eoff + 16 * t, 16)] for t in range(4)]
                    prod = (qg[i, pl.ds(0, 16)] * (kvg[i, pl.ds(0, 16)] + ev[0])
                            + qg[i, pl.ds(16, 16)] * (kvg[i, pl.ds(16, 16)] + ev[1])
                            + qg[i, pl.ds(32, 16)] * (kvg[i, pl.ds(32, 16)] + ev[2])
                            + qg[i, pl.ds(48, 16)] * (kvg[i, pl.ds(48, 16)] + ev[3]))
                    for sh in (8, 4, 2, 1):
                        idx = lane ^ sh
                        prod = prod + _lane_shuffle(prod, idx)
                    ew = jnp.exp(prod)
                    for t in range(4):
                        sbuf[i, pl.ds(16 * t, 16)] = ew * (
                            kvg[i, pl.ds(D + 16 * t, 16)] + ev[t])
                    sbuf[i, pl.ds(D, 16)] = ew * onehot0

            pltpu.sync_copy(sbuf, acc.at[dst2d.at[b]], add=True)

        plsc.subcore_barrier()

        @pl.loop(0, NCH_LOOP)
        def _copy_out(jc):
            m = s + jc * NS
            @pl.when(m < NCHT)
            def _():
                off = m * RCH
                pltpu.sync_copy(acc.at[pl.ds(off, RCH)], zbuf)
                pltpu.sync_copy(zbuf, out_hbm.at[pl.ds(c * N_NODES + off, RCH)])

    return sc_kernel(q, kv, e2, src3, dst3)


def kernel(x, edge_index, edge_attr, Wq0, bq0, Wk0, bk0, Wv0, bv0, We0, be0,
           Wq1, bq1, Wk1, bk1, Wv1, bv1, We1, be1):
    src3 = edge_index[0].reshape(NW, NBLK, EB)
    dst3 = edge_index[1].reshape(NW, NBLK, EB)
    q0, kv0 = _proj(x, Wq0, bq0, Wk0, bk0, Wv0, bv0)
    e0, e1 = _edge_embed(edge_attr, We0, be0, We1, be1)
    p0 = _sc_attention(q0, kv0, e0.reshape(N_EDGES // 2, 2 * D), src3, dst3)
    q1, kv1 = _mid(p0, Wq1, bq1, Wk1, bk1, Wv1, bv1)
    p1 = _sc_attention(q1, kv1, e1.reshape(N_EDGES // 2, 2 * D), src3, dst3)
    return _final(p1)


# speedup vs baseline: 9.7627x; 1.2322x over previous
import functools

import jax
import jax.numpy as jnp
from jax import lax
from jax.experimental import pallas as pl
from jax.experimental.pallas import tpu as pltpu
from jax.experimental.pallas import tpu_sc as plsc

N_NODES = 10000
N_EDGES = 320000
D_IN = 128
D_EDGE = 16
D = 64

NC = 2
NS = 16
NW = NC * NS
EPW = N_EDGES // NW
EB = 80
NBLK = EPW // EB
ACC_W = 80
RCH = 80
NCHT = N_NODES // RCH
NCH_LOOP = -(-NCHT // NS)


def _proj_body(x_ref, wq_ref, bq_ref, wk_ref, bk_ref, wv_ref, bv_ref,
               q_ref, kv_ref):
    xb = x_ref[...]
    q_ref[...] = (jnp.dot(xb, wq_ref[...], preferred_element_type=jnp.float32)
                  + bq_ref[...]) * 0.125
    kb = jnp.dot(xb, wk_ref[...], preferred_element_type=jnp.float32) + bk_ref[...]
    vb = jnp.dot(xb, wv_ref[...], preferred_element_type=jnp.float32) + bv_ref[...]
    kv_ref[...] = jnp.concatenate([kb, vb], axis=1)


def _proj(x, wq, bq, wk, bk, wv, bv):
    n, din = x.shape
    rb = 2000
    w_spec = pl.BlockSpec((din, D), lambda i: (0, 0))
    b_spec = pl.BlockSpec((1, D), lambda i: (0, 0))
    return pl.pallas_call(
        _proj_body,
        grid=(n // rb,),
        in_specs=[pl.BlockSpec((rb, din), lambda i: (i, 0)),
                  w_spec, b_spec, w_spec, b_spec, w_spec, b_spec],
        out_specs=[pl.BlockSpec((rb, D), lambda i: (i, 0)),
                   pl.BlockSpec((rb, 2 * D), lambda i: (i, 0))],
        out_shape=[jax.ShapeDtypeStruct((n, D), jnp.float32),
                   jax.ShapeDtypeStruct((n, 2 * D), jnp.float32)],
    )(x, wq, bq.reshape(1, D), wk, bk.reshape(1, D), wv, bv.reshape(1, D))


def _edge_body(a_ref, w0_ref, b0_ref, w1_ref, b1_ref, e0_ref, e1_ref):
    ab = a_ref[...]
    e0_ref[...] = jnp.dot(ab, w0_ref[...], preferred_element_type=jnp.float32) + b0_ref[...]
    e1_ref[...] = jnp.dot(ab, w1_ref[...], preferred_element_type=jnp.float32) + b1_ref[...]


def _edge_embed(edge_attr, w0, b0, w1, b1):
    rb = 8000
    w_spec = pl.BlockSpec((D_EDGE, D), lambda i: (0, 0))
    b_spec = pl.BlockSpec((1, D), lambda i: (0, 0))
    return pl.pallas_call(
        _edge_body,
        grid=(N_EDGES // rb,),
        in_specs=[pl.BlockSpec((rb, D_EDGE), lambda i: (i, 0)),
                  w_spec, b_spec, w_spec, b_spec],
        out_specs=[pl.BlockSpec((rb, D), lambda i: (i, 0))] * 2,
        out_shape=[jax.ShapeDtypeStruct((N_EDGES, D), jnp.float32)] * 2,
    )(edge_attr, w0, b0.reshape(1, D), w1, b1.reshape(1, D))


def _mid_body(p0_ref, p1_ref, wq_ref, bq_ref, wk_ref, bk_ref, wv_ref, bv_ref,
              q_ref, kv_ref):
    ps = p0_ref[...] + p1_ref[...]
    num = ps[:, :D]
    den = ps[:, D:D + 1]
    h = jnp.maximum(num / (den + 1e-16), 0.0)
    q_ref[...] = (jnp.dot(h, wq_ref[...], preferred_element_type=jnp.float32)
                  + bq_ref[...]) * 0.125
    kb = jnp.dot(h, wk_ref[...], preferred_element_type=jnp.float32) + bk_ref[...]
    vb = jnp.dot(h, wv_ref[...], preferred_element_type=jnp.float32) + bv_ref[...]
    kv_ref[...] = jnp.concatenate([kb, vb], axis=1)


def _mid(p, wq, bq, wk, bk, wv, bv):
    rb = 2000
    nb = N_NODES // rb
    w_spec = pl.BlockSpec((D, D), lambda i: (0, 0))
    b_spec = pl.BlockSpec((1, D), lambda i: (0, 0))
    return pl.pallas_call(
        _mid_body,
        grid=(nb,),
        in_specs=[pl.BlockSpec((rb, ACC_W), lambda i: (i, 0)),
                  pl.BlockSpec((rb, ACC_W), lambda i: (i + nb, 0)),
                  w_spec, b_spec, w_spec, b_spec, w_spec, b_spec],
        out_specs=[pl.BlockSpec((rb, D), lambda i: (i, 0)),
                   pl.BlockSpec((rb, 2 * D), lambda i: (i, 0))],
        out_shape=[jax.ShapeDtypeStruct((N_NODES, D), jnp.float32),
                   jax.ShapeDtypeStruct((N_NODES, 2 * D), jnp.float32)],
    )(p, p, wq, bq.reshape(1, D), wk, bk.reshape(1, D), wv, bv.reshape(1, D))


def _final_body(p0_ref, p1_ref, o_ref):
    ps = p0_ref[...] + p1_ref[...]
    o_ref[...] = ps[:, :D] / (ps[:, D:D + 1] + 1e-16)


def _final(p):
    rb = 2000
    nb = N_NODES // rb
    return pl.pallas_call(
        _final_body,
        grid=(nb,),
        in_specs=[pl.BlockSpec((rb, ACC_W), lambda i: (i, 0)),
                  pl.BlockSpec((rb, ACC_W), lambda i: (i + nb, 0))],
        out_specs=pl.BlockSpec((rb, D), lambda i: (i, 0)),
        out_shape=jax.ShapeDtypeStruct((N_NODES, D), jnp.float32),
    )(p, p)


_GATHER_DNUMS = lax.GatherDimensionNumbers(
    offset_dims=(), collapsed_slice_dims=(0,), start_index_map=(0,))


def _lane_shuffle(vv, idx):
    return lax.gather(vv, idx[:, None], _GATHER_DNUMS, (1,),
                      mode=lax.GatherScatterMode.PROMISE_IN_BOUNDS)

def _sc_attention(q, kv, e2, src3, dst3):
    mesh = plsc.VectorSubcoreMesh(core_axis_name="c", subcore_axis_name="s")

    @functools.partial(
        pl.kernel,
        out_type=jax.ShapeDtypeStruct((NC * N_NODES, ACC_W), jnp.float32),
        mesh=mesh,
        scratch_types=[
            pltpu.VMEM((NBLK, EB), jnp.int32),
            pltpu.VMEM((NBLK, EB), jnp.int32),
            pltpu.VMEM((EB, D), jnp.float32),
            pltpu.VMEM((EB, 2 * D), jnp.float32),
            pltpu.VMEM((EB // 2, 2 * D), jnp.float32),
            pltpu.VMEM((EB, D), jnp.float32),
            pltpu.VMEM((EB, 2 * D), jnp.float32),
            pltpu.VMEM((EB // 2, 2 * D), jnp.float32),
            pltpu.VMEM((EB, ACC_W), jnp.float32),
            pltpu.VMEM((RCH, ACC_W), jnp.float32),
            pltpu.VMEM_SHARED((N_NODES, ACC_W), jnp.float32),
            pltpu.SemaphoreType.DMA,
            pltpu.SemaphoreType.DMA,
            pltpu.SemaphoreType.DMA,
            pltpu.SemaphoreType.DMA,
            pltpu.SemaphoreType.DMA,
            pltpu.SemaphoreType.DMA,
        ],
        compiler_params=pltpu.CompilerParams(use_tc_tiling_on_sc=False),
    )
    def sc_kernel(q_hbm, kv_hbm, e_hbm, src_hbm, dst_hbm, out_hbm,
                  src2d, dst2d, qg0, kvg0, eg0, qg1, kvg1, eg1, sbuf, zbuf, acc,
                  sq0, skv0, se0, sq1, skv1, se1):
        c = lax.axis_index("c")
        s = lax.axis_index("s")
        wid = c * NS + s

        @pl.loop(0, RCH)
        def _zero_rows(i):
            for j in range(ACC_W // 16):
                zbuf[i, pl.ds(j * 16, 16)] = jnp.zeros((16,), jnp.float32)

        @pl.loop(0, NCH_LOOP)
        def _zero_acc(jc):
            m = s + jc * NS
            @pl.when(m < NCHT)
            def _():
                pltpu.sync_copy(zbuf, acc.at[pl.ds(m * RCH, RCH)])

        pltpu.sync_copy(src_hbm.at[wid], src2d)
        pltpu.sync_copy(dst_hbm.at[wid], dst2d)

        plsc.subcore_barrier()

        lane = lax.iota(jnp.int32, 16)
        onehot0 = jnp.where(lane == 0, 1.0, 0.0).astype(jnp.float32)
        ebase0 = wid * (EPW // 2)

        slots = ((qg0, kvg0, eg0, sq0, skv0, se0),
                 (qg1, kvg1, eg1, sq1, skv1, se1))

        def issue(b, slot):
            qg, kvg, eg, sq, skv, se = slots[slot]
            pltpu.async_copy(q_hbm.at[dst2d.at[b]], qg, sq)
            pltpu.async_copy(kv_hbm.at[src2d.at[b]], kvg, skv)
            pltpu.async_copy(
                e_hbm.at[pl.ds(ebase0 + b * (EB // 2), EB // 2)], eg, se)

        def wait(slot):
            qg, kvg, eg, sq, skv, se = slots[slot]
            pltpu.make_async_copy(q_hbm.at[dst2d.at[0]], qg, sq).wait()
            pltpu.make_async_copy(kv_hbm.at[src2d.at[0]], kvg, skv).wait()
            pltpu.make_async_copy(
                e_hbm.at[pl.ds(0, EB // 2)], eg, se).wait()

        def compute_and_scatter(b, slot):
            qg, kvg, eg, _, _, _ = slots[slot]

            @pl.loop(0, EB // 2)
            def _edge_pairs(j):
                for half in range(2):
                    i = 2 * j + half
                    eoff = half * D
                    ev = [eg[j, pl.ds(eoff + 16 * t, 16)] for t in range(4)]
                    prod = (qg[i, pl.ds(0, 16)] * (kvg[i, pl.ds(0, 16)] + ev[0])
                            + qg[i, pl.ds(16, 16)] * (kvg[i, pl.ds(16, 16)] + ev[1])
                            + qg[i, pl.ds(32, 16)] * (kvg[i, pl.ds(32, 16)] + ev[2])
                            + qg[i, pl.ds(48, 16)] * (kvg[i, pl.ds(48, 16)] + ev[3]))
                    for sh in (8, 4, 2, 1):
                        idx = lane ^ sh
                        prod = prod + _lane_shuffle(prod, idx)
                    ew = jnp.exp(prod)
                    for t in range(4):
                        sbuf[i, pl.ds(16 * t, 16)] = ew * (
                            kvg[i, pl.ds(D + 16 * t, 16)] + ev[t])
                    sbuf[i, pl.ds(D, 16)] = ew * onehot0

            pltpu.sync_copy(sbuf, acc.at[dst2d.at[b]], add=True)

        issue(0, 0)

        @pl.loop(0, (NBLK + 1) // 2)
        def _pairs(p):
            b0 = 2 * p
            b1 = b0 + 1

            @pl.when(b1 < NBLK)
            def _():
                issue(b1, 1)

            wait(0)
            compute_and_scatter(b0, 0)

            @pl.when(b1 < NBLK)
            def _():
                @pl.when(b1 + 1 < NBLK)
                def _():
                    issue(b1 + 1, 0)

                wait(1)
                compute_and_scatter(b1, 1)

        plsc.subcore_barrier()

        @pl.loop(0, NCH_LOOP)
        def _copy_out(jc):
            m = s + jc * NS
            @pl.when(m < NCHT)
            def _():
                off = m * RCH
                pltpu.sync_copy(acc.at[pl.ds(off, RCH)], zbuf)
                pltpu.sync_copy(zbuf, out_hbm.at[pl.ds(c * N_NODES + off, RCH)])

    return sc_kernel(q, kv, e2, src3, dst3)


def kernel(x, edge_index, edge_attr, Wq0, bq0, Wk0, bk0, Wv0, bv0, We0, be0,
           Wq1, bq1, Wk1, bk1, Wv1, bv1, We1, be1):
    src3 = edge_index[0].reshape(NW, NBLK, EB)
    dst3 = edge_index[1].reshape(NW, NBLK, EB)
    q0, kv0 = _proj(x, Wq0, bq0, Wk0, bk0, Wv0, bv0)
    e0, e1 = _edge_embed(edge_attr, We0, be0, We1, be1)
    p0 = _sc_attention(q0, kv0, e0.reshape(N_EDGES // 2, 2 * D), src3, dst3)
    q1, kv1 = _mid(p0, Wq1, bq1, Wk1, bk1, Wv1, bv1)
    p1 = _sc_attention(q1, kv1, e1.reshape(N_EDGES // 2, 2 * D), src3, dst3)
    return _final(p1)


# speedup vs baseline: 10.2497x; 1.0499x over previous
import functools

import jax
import jax.numpy as jnp
from jax import lax
from jax.experimental import pallas as pl
from jax.experimental.pallas import tpu as pltpu
from jax.experimental.pallas import tpu_sc as plsc

N_NODES = 10000
N_EDGES = 320000
D_IN = 128
D_EDGE = 16
D = 64

NC = 2
NS = 16
NW = NC * NS
EPW = N_EDGES // NW
EB = 80
NBLK = EPW // EB
ACC_W = 80
RCH = 80
NCHT = N_NODES // RCH
NCH_LOOP = -(-NCHT // NS)


def _proj_body(x_ref, wq_ref, bq_ref, wk_ref, bk_ref, wv_ref, bv_ref,
               q_ref, kv_ref):
    xb = x_ref[...]
    q_ref[...] = (jnp.dot(xb, wq_ref[...], preferred_element_type=jnp.float32)
                  + bq_ref[...]) * 0.125
    kb = jnp.dot(xb, wk_ref[...], preferred_element_type=jnp.float32) + bk_ref[...]
    vb = jnp.dot(xb, wv_ref[...], preferred_element_type=jnp.float32) + bv_ref[...]
    kv_ref[...] = jnp.concatenate([kb, vb], axis=1)


def _proj(x, wq, bq, wk, bk, wv, bv):
    n, din = x.shape
    rb = 2000
    w_spec = pl.BlockSpec((din, D), lambda i: (0, 0))
    b_spec = pl.BlockSpec((1, D), lambda i: (0, 0))
    return pl.pallas_call(
        _proj_body,
        grid=(n // rb,),
        in_specs=[pl.BlockSpec((rb, din), lambda i: (i, 0)),
                  w_spec, b_spec, w_spec, b_spec, w_spec, b_spec],
        out_specs=[pl.BlockSpec((rb, D), lambda i: (i, 0)),
                   pl.BlockSpec((rb, 2 * D), lambda i: (i, 0))],
        out_shape=[jax.ShapeDtypeStruct((n, D), jnp.float32),
                   jax.ShapeDtypeStruct((n, 2 * D), jnp.float32)],
    )(x, wq, bq.reshape(1, D), wk, bk.reshape(1, D), wv, bv.reshape(1, D))


def _edge_body(a_ref, w0_ref, b0_ref, w1_ref, b1_ref, e0_ref, e1_ref):
    ab = a_ref[...]
    e0_ref[...] = jnp.dot(ab, w0_ref[...], preferred_element_type=jnp.float32) + b0_ref[...]
    e1_ref[...] = jnp.dot(ab, w1_ref[...], preferred_element_type=jnp.float32) + b1_ref[...]


def _edge_embed(edge_attr, w0, b0, w1, b1):
    rb = 8000
    w_spec = pl.BlockSpec((D_EDGE, D), lambda i: (0, 0))
    b_spec = pl.BlockSpec((1, D), lambda i: (0, 0))
    return pl.pallas_call(
        _edge_body,
        grid=(N_EDGES // rb,),
        in_specs=[pl.BlockSpec((rb, D_EDGE), lambda i: (i, 0)),
                  w_spec, b_spec, w_spec, b_spec],
        out_specs=[pl.BlockSpec((rb, D), lambda i: (i, 0))] * 2,
        out_shape=[jax.ShapeDtypeStruct((N_EDGES, D), jnp.float32)] * 2,
    )(edge_attr, w0, b0.reshape(1, D), w1, b1.reshape(1, D))


def _mid_body(p0_ref, p1_ref, wq_ref, bq_ref, wk_ref, bk_ref, wv_ref, bv_ref,
              q_ref, kv_ref):
    ps = p0_ref[...] + p1_ref[...]
    num = ps[:, :D]
    den = ps[:, D:D + 1]
    h = jnp.maximum(num / (den + 1e-16), 0.0)
    q_ref[...] = (jnp.dot(h, wq_ref[...], preferred_element_type=jnp.float32)
                  + bq_ref[...]) * 0.125
    kb = jnp.dot(h, wk_ref[...], preferred_element_type=jnp.float32) + bk_ref[...]
    vb = jnp.dot(h, wv_ref[...], preferred_element_type=jnp.float32) + bv_ref[...]
    kv_ref[...] = jnp.concatenate([kb, vb], axis=1)


def _mid(p, wq, bq, wk, bk, wv, bv):
    rb = 2000
    nb = N_NODES // rb
    w_spec = pl.BlockSpec((D, D), lambda i: (0, 0))
    b_spec = pl.BlockSpec((1, D), lambda i: (0, 0))
    return pl.pallas_call(
        _mid_body,
        grid=(nb,),
        in_specs=[pl.BlockSpec((rb, ACC_W), lambda i: (i, 0)),
                  pl.BlockSpec((rb, ACC_W), lambda i: (i + nb, 0)),
                  w_spec, b_spec, w_spec, b_spec, w_spec, b_spec],
        out_specs=[pl.BlockSpec((rb, D), lambda i: (i, 0)),
                   pl.BlockSpec((rb, 2 * D), lambda i: (i, 0))],
        out_shape=[jax.ShapeDtypeStruct((N_NODES, D), jnp.float32),
                   jax.ShapeDtypeStruct((N_NODES, 2 * D), jnp.float32)],
    )(p, p, wq, bq.reshape(1, D), wk, bk.reshape(1, D), wv, bv.reshape(1, D))


def _final_body(p0_ref, p1_ref, o_ref):
    ps = p0_ref[...] + p1_ref[...]
    o_ref[...] = ps[:, :D] / (ps[:, D:D + 1] + 1e-16)


def _final(p):
    rb = 2000
    nb = N_NODES // rb
    return pl.pallas_call(
        _final_body,
        grid=(nb,),
        in_specs=[pl.BlockSpec((rb, ACC_W), lambda i: (i, 0)),
                  pl.BlockSpec((rb, ACC_W), lambda i: (i + nb, 0))],
        out_specs=pl.BlockSpec((rb, D), lambda i: (i, 0)),
        out_shape=jax.ShapeDtypeStruct((N_NODES, D), jnp.float32),
    )(p, p)


_GATHER_DNUMS = lax.GatherDimensionNumbers(
    offset_dims=(), collapsed_slice_dims=(0,), start_index_map=(0,))


def _lane_shuffle(vv, idx):
    return lax.gather(vv, idx[:, None], _GATHER_DNUMS, (1,),
                      mode=lax.GatherScatterMode.PROMISE_IN_BOUNDS)

def _sc_attention(q, kv, e2, src3, dst3):
    mesh = plsc.VectorSubcoreMesh(core_axis_name="c", subcore_axis_name="s")

    @functools.partial(
        pl.kernel,
        out_type=jax.ShapeDtypeStruct((NC * N_NODES, ACC_W), jnp.float32),
        mesh=mesh,
        scratch_types=[
            pltpu.VMEM((NBLK, EB), jnp.int32),
            pltpu.VMEM((NBLK, EB), jnp.int32),
            pltpu.VMEM((EB, D), jnp.float32),
            pltpu.VMEM((EB, 2 * D), jnp.float32),
            pltpu.VMEM((EB // 2, 2 * D), jnp.float32),
            pltpu.VMEM((EB, D), jnp.float32),
            pltpu.VMEM((EB, 2 * D), jnp.float32),
            pltpu.VMEM((EB // 2, 2 * D), jnp.float32),
            pltpu.VMEM((EB, ACC_W), jnp.float32),
            pltpu.VMEM((EB, ACC_W), jnp.float32),
            pltpu.VMEM((RCH, ACC_W), jnp.float32),
            pltpu.VMEM_SHARED((N_NODES, ACC_W), jnp.float32),
            pltpu.SemaphoreType.DMA,
            pltpu.SemaphoreType.DMA,
            pltpu.SemaphoreType.DMA,
            pltpu.SemaphoreType.DMA,
            pltpu.SemaphoreType.DMA,
            pltpu.SemaphoreType.DMA,
            pltpu.SemaphoreType.DMA,
            pltpu.SemaphoreType.DMA,
        ],
        compiler_params=pltpu.CompilerParams(use_tc_tiling_on_sc=False),
    )
    def sc_kernel(q_hbm, kv_hbm, e_hbm, src_hbm, dst_hbm, out_hbm,
                  src2d, dst2d, qg0, kvg0, eg0, qg1, kvg1, eg1,
                  sbuf0, sbuf1, zbuf, acc,
                  sq0, skv0, se0, sq1, skv1, se1, ssc0, ssc1):
        c = lax.axis_index("c")
        s = lax.axis_index("s")
        wid = c * NS + s

        @pl.loop(0, RCH)
        def _zero_rows(i):
            for j in range(ACC_W // 16):
                zbuf[i, pl.ds(j * 16, 16)] = jnp.zeros((16,), jnp.float32)

        @pl.loop(0, NCH_LOOP)
        def _zero_acc(jc):
            m = s + jc * NS
            @pl.when(m < NCHT)
            def _():
                pltpu.sync_copy(zbuf, acc.at[pl.ds(m * RCH, RCH)])

        pltpu.sync_copy(src_hbm.at[wid], src2d)
        pltpu.sync_copy(dst_hbm.at[wid], dst2d)

        plsc.subcore_barrier()

        lane = lax.iota(jnp.int32, 16)
        onehot0 = jnp.where(lane == 0, 1.0, 0.0).astype(jnp.float32)
        ebase0 = wid * (EPW // 2)

        slots = ((qg0, kvg0, eg0, sq0, skv0, se0, sbuf0, ssc0),
                 (qg1, kvg1, eg1, sq1, skv1, se1, sbuf1, ssc1))

        def issue(b, slot):
            qg, kvg, eg, sq, skv, se, _, _ = slots[slot]
            pltpu.async_copy(q_hbm.at[dst2d.at[b]], qg, sq)
            pltpu.async_copy(kv_hbm.at[src2d.at[b]], kvg, skv)
            pltpu.async_copy(
                e_hbm.at[pl.ds(ebase0 + b * (EB // 2), EB // 2)], eg, se)

        def wait(slot):
            qg, kvg, eg, sq, skv, se, _, _ = slots[slot]
            pltpu.make_async_copy(q_hbm.at[dst2d.at[0]], qg, sq).wait()
            pltpu.make_async_copy(kv_hbm.at[src2d.at[0]], kvg, skv).wait()
            pltpu.make_async_copy(
                e_hbm.at[pl.ds(0, EB // 2)], eg, se).wait()

        def compute_and_scatter(b, slot):
            qg, kvg, eg, _, _, _, sbuf, ssc = slots[slot]

            @pl.when(b >= 2)
            def _():
                pltpu.make_async_copy(sbuf, acc.at[dst2d.at[0]], ssc).wait()

            @pl.loop(0, EB // 2)
            def _edge_pairs(j):
                for half in range(2):
                    i = 2 * j + half
                    eoff = half * D
                    ev = [eg[j, pl.ds(eoff + 16 * t, 16)] for t in range(4)]
                    prod = (qg[i, pl.ds(0, 16)] * (kvg[i, pl.ds(0, 16)] + ev[0])
                            + qg[i, pl.ds(16, 16)] * (kvg[i, pl.ds(16, 16)] + ev[1])
                            + qg[i, pl.ds(32, 16)] * (kvg[i, pl.ds(32, 16)] + ev[2])
                            + qg[i, pl.ds(48, 16)] * (kvg[i, pl.ds(48, 16)] + ev[3]))
                    for sh in (8, 4, 2, 1):
                        idx = lane ^ sh
                        prod = prod + _lane_shuffle(prod, idx)
                    ew = jnp.exp(prod)
                    for t in range(4):
                        sbuf[i, pl.ds(16 * t, 16)] = ew * (
                            kvg[i, pl.ds(D + 16 * t, 16)] + ev[t])
                    sbuf[i, pl.ds(D, 16)] = ew * onehot0

            pltpu.async_copy(sbuf, acc.at[dst2d.at[b]], ssc, add=True)

        issue(0, 0)

        @pl.loop(0, (NBLK + 1) // 2)
        def _pairs(p):
            b0 = 2 * p
            b1 = b0 + 1

            @pl.when(b1 < NBLK)
            def _():
                issue(b1, 1)

            wait(0)
            compute_and_scatter(b0, 0)

            @pl.when(b1 < NBLK)
            def _():
                @pl.when(b1 + 1 < NBLK)
                def _():
                    issue(b1 + 1, 0)

                wait(1)
                compute_and_scatter(b1, 1)

        pltpu.make_async_copy(sbuf0, acc.at[dst2d.at[0]], ssc0).wait()
        pltpu.make_async_copy(sbuf1, acc.at[dst2d.at[0]], ssc1).wait()

        plsc.subcore_barrier()

        @pl.loop(0, NCH_LOOP)
        def _copy_out(jc):
            m = s + jc * NS
            @pl.when(m < NCHT)
            def _():
                off = m * RCH
                pltpu.sync_copy(acc.at[pl.ds(off, RCH)], zbuf)
                pltpu.sync_copy(zbuf, out_hbm.at[pl.ds(c * N_NODES + off, RCH)])

    return sc_kernel(q, kv, e2, src3, dst3)


def kernel(x, edge_index, edge_attr, Wq0, bq0, Wk0, bk0, Wv0, bv0, We0, be0,
           Wq1, bq1, Wk1, bk1, Wv1, bv1, We1, be1):
    src3 = edge_index[0].reshape(NW, NBLK, EB)
    dst3 = edge_index[1].reshape(NW, NBLK, EB)
    q0, kv0 = _proj(x, Wq0, bq0, Wk0, bk0, Wv0, bv0)
    e0, e1 = _edge_embed(edge_attr, We0, be0, We1, be1)
    p0 = _sc_attention(q0, kv0, e0.reshape(N_EDGES // 2, 2 * D), src3, dst3)
    q1, kv1 = _mid(p0, Wq1, bq1, Wk1, bk1, Wv1, bv1)
    p1 = _sc_attention(q1, kv1, e1.reshape(N_EDGES // 2, 2 * D), src3, dst3)
    return _final(p1)


# speedup vs baseline: 11.4146x; 1.1137x over previous
import functools

import jax
import jax.numpy as jnp
from jax import lax
from jax.experimental import pallas as pl
from jax.experimental.pallas import tpu as pltpu
from jax.experimental.pallas import tpu_sc as plsc

N_NODES = 10000
N_EDGES = 320000
D_IN = 128
D_EDGE = 16
D = 64

NC = 2
NS = 16
NW = NC * NS
EPW = N_EDGES // NW
EB = 80
NBLK = EPW // EB
ACC_W = 80
RCH = 80
NCHT = N_NODES // RCH
NCH_LOOP = -(-NCHT // NS)


def _proj_body(x_ref, wq_ref, bq_ref, wk_ref, bk_ref, wv_ref, bv_ref,
               q_ref, kv_ref):
    xb = x_ref[...]
    q_ref[...] = (jnp.dot(xb, wq_ref[...], preferred_element_type=jnp.float32)
                  + bq_ref[...]) * 0.125
    kb = jnp.dot(xb, wk_ref[...], preferred_element_type=jnp.float32) + bk_ref[...]
    vb = jnp.dot(xb, wv_ref[...], preferred_element_type=jnp.float32) + bv_ref[...]
    kv_ref[...] = jnp.concatenate([kb, vb], axis=1)


def _proj(x, wq, bq, wk, bk, wv, bv):
    n, din = x.shape
    rb = 2000
    w_spec = pl.BlockSpec((din, D), lambda i: (0, 0))
    b_spec = pl.BlockSpec((1, D), lambda i: (0, 0))
    return pl.pallas_call(
        _proj_body,
        grid=(n // rb,),
        in_specs=[pl.BlockSpec((rb, din), lambda i: (i, 0)),
                  w_spec, b_spec, w_spec, b_spec, w_spec, b_spec],
        out_specs=[pl.BlockSpec((rb, D), lambda i: (i, 0)),
                   pl.BlockSpec((rb, 2 * D), lambda i: (i, 0))],
        out_shape=[jax.ShapeDtypeStruct((n, D), jnp.float32),
                   jax.ShapeDtypeStruct((n, 2 * D), jnp.float32)],
    )(x, wq, bq.reshape(1, D), wk, bk.reshape(1, D), wv, bv.reshape(1, D))


def _edge_body(a_ref, w0_ref, b0_ref, w1_ref, b1_ref, e0_ref, e1_ref):
    ab = a_ref[...]
    e0_ref[...] = jnp.dot(ab, w0_ref[...], preferred_element_type=jnp.float32) + b0_ref[...]
    e1_ref[...] = jnp.dot(ab, w1_ref[...], preferred_element_type=jnp.float32) + b1_ref[...]


def _edge_embed(edge_attr, w0, b0, w1, b1):
    ea2 = edge_attr.reshape(N_EDGES // 2, 2 * D_EDGE)
    z = jnp.zeros((D_EDGE, D), jnp.float32)
    w20 = jnp.concatenate(
        [jnp.concatenate([w0, z], axis=1), jnp.concatenate([z, w0], axis=1)],
        axis=0)
    w21 = jnp.concatenate(
        [jnp.concatenate([w1, z], axis=1), jnp.concatenate([z, w1], axis=1)],
        axis=0)
    b20 = jnp.concatenate([b0, b0]).reshape(1, 2 * D)
    b21 = jnp.concatenate([b1, b1]).reshape(1, 2 * D)
    rb = 4000
    w_spec = pl.BlockSpec((2 * D_EDGE, 2 * D), lambda i: (0, 0))
    b_spec = pl.BlockSpec((1, 2 * D), lambda i: (0, 0))
    return pl.pallas_call(
        _edge_body,
        grid=(N_EDGES // 2 // rb,),
        in_specs=[pl.BlockSpec((rb, 2 * D_EDGE), lambda i: (i, 0)),
                  w_spec, b_spec, w_spec, b_spec],
        out_specs=[pl.BlockSpec((rb, 2 * D), lambda i: (i, 0))] * 2,
        out_shape=[jax.ShapeDtypeStruct((N_EDGES // 2, 2 * D), jnp.float32)] * 2,
    )(ea2, w20, b20, w21, b21)


def _mid_body(p0_ref, p1_ref, wq_ref, bq_ref, wk_ref, bk_ref, wv_ref, bv_ref,
              q_ref, kv_ref):
    ps = p0_ref[...] + p1_ref[...]
    num = ps[:, :D]
    den = ps[:, D:D + 1]
    h = jnp.maximum(num / (den + 1e-16), 0.0)
    q_ref[...] = (jnp.dot(h, wq_ref[...], preferred_element_type=jnp.float32)
                  + bq_ref[...]) * 0.125
    kb = jnp.dot(h, wk_ref[...], preferred_element_type=jnp.float32) + bk_ref[...]
    vb = jnp.dot(h, wv_ref[...], preferred_element_type=jnp.float32) + bv_ref[...]
    kv_ref[...] = jnp.concatenate([kb, vb], axis=1)


def _mid(p, wq, bq, wk, bk, wv, bv):
    rb = 2000
    nb = N_NODES // rb
    w_spec = pl.BlockSpec((D, D), lambda i: (0, 0))
    b_spec = pl.BlockSpec((1, D), lambda i: (0, 0))
    return pl.pallas_call(
        _mid_body,
        grid=(nb,),
        in_specs=[pl.BlockSpec((rb, ACC_W), lambda i: (i, 0)),
                  pl.BlockSpec((rb, ACC_W), lambda i: (i + nb, 0)),
                  w_spec, b_spec, w_spec, b_spec, w_spec, b_spec],
        out_specs=[pl.BlockSpec((rb, D), lambda i: (i, 0)),
                   pl.BlockSpec((rb, 2 * D), lambda i: (i, 0))],
        out_shape=[jax.ShapeDtypeStruct((N_NODES, D), jnp.float32),
                   jax.ShapeDtypeStruct((N_NODES, 2 * D), jnp.float32)],
    )(p, p, wq, bq.reshape(1, D), wk, bk.reshape(1, D), wv, bv.reshape(1, D))


def _final_body(p0_ref, p1_ref, o_ref):
    ps = p0_ref[...] + p1_ref[...]
    o_ref[...] = ps[:, :D] / (ps[:, D:D + 1] + 1e-16)


def _final(p):
    rb = 2000
    nb = N_NODES // rb
    return pl.pallas_call(
        _final_body,
        grid=(nb,),
        in_specs=[pl.BlockSpec((rb, ACC_W), lambda i: (i, 0)),
                  pl.BlockSpec((rb, ACC_W), lambda i: (i + nb, 0))],
        out_specs=pl.BlockSpec((rb, D), lambda i: (i, 0)),
        out_shape=jax.ShapeDtypeStruct((N_NODES, D), jnp.float32),
    )(p, p)


_GATHER_DNUMS = lax.GatherDimensionNumbers(
    offset_dims=(), collapsed_slice_dims=(0,), start_index_map=(0,))


def _lane_shuffle(vv, idx):
    return lax.gather(vv, idx[:, None], _GATHER_DNUMS, (1,),
                      mode=lax.GatherScatterMode.PROMISE_IN_BOUNDS)

def _sc_attention(q, kv, e2, src3, dst3):
    mesh = plsc.VectorSubcoreMesh(core_axis_name="c", subcore_axis_name="s")

    @functools.partial(
        pl.kernel,
        out_type=jax.ShapeDtypeStruct((NC * N_NODES, ACC_W), jnp.float32),
        mesh=mesh,
        scratch_types=[
            pltpu.VMEM((NBLK, EB), jnp.int32),
            pltpu.VMEM((NBLK, EB), jnp.int32),
            pltpu.VMEM((EB, D), jnp.float32),
            pltpu.VMEM((EB, 2 * D), jnp.float32),
            pltpu.VMEM((EB // 2, 2 * D), jnp.float32),
            pltpu.VMEM((EB, D), jnp.float32),
            pltpu.VMEM((EB, 2 * D), jnp.float32),
            pltpu.VMEM((EB // 2, 2 * D), jnp.float32),
            pltpu.VMEM((EB, ACC_W), jnp.float32),
            pltpu.VMEM((EB, ACC_W), jnp.float32),
            pltpu.VMEM((RCH, ACC_W), jnp.float32),
            pltpu.VMEM_SHARED((N_NODES, ACC_W), jnp.float32),
            pltpu.SemaphoreType.DMA,
            pltpu.SemaphoreType.DMA,
            pltpu.SemaphoreType.DMA,
            pltpu.SemaphoreType.DMA,
            pltpu.SemaphoreType.DMA,
            pltpu.SemaphoreType.DMA,
            pltpu.SemaphoreType.DMA,
            pltpu.SemaphoreType.DMA,
        ],
        compiler_params=pltpu.CompilerParams(use_tc_tiling_on_sc=False),
    )
    def sc_kernel(q_hbm, kv_hbm, e_hbm, src_hbm, dst_hbm, out_hbm,
                  src2d, dst2d, qg0, kvg0, eg0, qg1, kvg1, eg1,
                  sbuf0, sbuf1, zbuf, acc,
                  sq0, skv0, se0, sq1, skv1, se1, ssc0, ssc1):
        c = lax.axis_index("c")
        s = lax.axis_index("s")
        wid = c * NS + s

        @pl.loop(0, RCH)
        def _zero_rows(i):
            for j in range(ACC_W // 16):
                zbuf[i, pl.ds(j * 16, 16)] = jnp.zeros((16,), jnp.float32)

        @pl.loop(0, NCH_LOOP)
        def _zero_acc(jc):
            m = s + jc * NS
            @pl.when(m < NCHT)
            def _():
                pltpu.sync_copy(zbuf, acc.at[pl.ds(m * RCH, RCH)])

        pltpu.sync_copy(src_hbm.at[wid], src2d)
        pltpu.sync_copy(dst_hbm.at[wid], dst2d)

        plsc.subcore_barrier()

        lane = lax.iota(jnp.int32, 16)
        onehot0 = jnp.where(lane == 0, 1.0, 0.0).astype(jnp.float32)
        ebase0 = wid * (EPW // 2)

        slots = ((qg0, kvg0, eg0, sq0, skv0, se0, sbuf0, ssc0),
                 (qg1, kvg1, eg1, sq1, skv1, se1, sbuf1, ssc1))

        def issue(b, slot):
            qg, kvg, eg, sq, skv, se, _, _ = slots[slot]
            pltpu.async_copy(q_hbm.at[dst2d.at[b]], qg, sq)
            pltpu.async_copy(kv_hbm.at[src2d.at[b]], kvg, skv)
            pltpu.async_copy(
                e_hbm.at[pl.ds(ebase0 + b * (EB // 2), EB // 2)], eg, se)

        def wait(slot):
            qg, kvg, eg, sq, skv, se, _, _ = slots[slot]
            pltpu.make_async_copy(q_hbm.at[dst2d.at[0]], qg, sq).wait()
            pltpu.make_async_copy(kv_hbm.at[src2d.at[0]], kvg, skv).wait()
            pltpu.make_async_copy(
                e_hbm.at[pl.ds(0, EB // 2)], eg, se).wait()

        def compute_and_scatter(b, slot):
            qg, kvg, eg, _, _, _, sbuf, ssc = slots[slot]

            @pl.when(b >= 2)
            def _():
                pltpu.make_async_copy(sbuf, acc.at[dst2d.at[0]], ssc).wait()

            @pl.loop(0, EB // 2)
            def _edge_pairs(j):
                for half in range(2):
                    i = 2 * j + half
                    eoff = half * D
                    ev = [eg[j, pl.ds(eoff + 16 * t, 16)] for t in range(4)]
                    prod = (qg[i, pl.ds(0, 16)] * (kvg[i, pl.ds(0, 16)] + ev[0])
                            + qg[i, pl.ds(16, 16)] * (kvg[i, pl.ds(16, 16)] + ev[1])
                            + qg[i, pl.ds(32, 16)] * (kvg[i, pl.ds(32, 16)] + ev[2])
                            + qg[i, pl.ds(48, 16)] * (kvg[i, pl.ds(48, 16)] + ev[3]))
                    for sh in (8, 4, 2, 1):
                        idx = lane ^ sh
                        prod = prod + _lane_shuffle(prod, idx)
                    ew = jnp.exp(prod)
                    for t in range(4):
                        sbuf[i, pl.ds(16 * t, 16)] = ew * (
                            kvg[i, pl.ds(D + 16 * t, 16)] + ev[t])
                    sbuf[i, pl.ds(D, 16)] = ew * onehot0

            pltpu.async_copy(sbuf, acc.at[dst2d.at[b]], ssc, add=True)

        issue(0, 0)

        @pl.loop(0, (NBLK + 1) // 2)
        def _pairs(p):
            b0 = 2 * p
            b1 = b0 + 1

            @pl.when(b1 < NBLK)
            def _():
                issue(b1, 1)

            wait(0)
            compute_and_scatter(b0, 0)

            @pl.when(b1 < NBLK)
            def _():
                @pl.when(b1 + 1 < NBLK)
                def _():
                    issue(b1 + 1, 0)

                wait(1)
                compute_and_scatter(b1, 1)

        pltpu.make_async_copy(sbuf0, acc.at[dst2d.at[0]], ssc0).wait()
        pltpu.make_async_copy(sbuf1, acc.at[dst2d.at[0]], ssc1).wait()

        plsc.subcore_barrier()

        @pl.loop(0, NCH_LOOP)
        def _copy_out(jc):
            m = s + jc * NS
            @pl.when(m < NCHT)
            def _():
                off = m * RCH
                pltpu.sync_copy(acc.at[pl.ds(off, RCH)], zbuf)
                pltpu.sync_copy(zbuf, out_hbm.at[pl.ds(c * N_NODES + off, RCH)])

    return sc_kernel(q, kv, e2, src3, dst3)


def kernel(x, edge_index, edge_attr, Wq0, bq0, Wk0, bk0, Wv0, bv0, We0, be0,
           Wq1, bq1, Wk1, bk1, Wv1, bv1, We1, be1):
    src3 = edge_index[0].reshape(NW, NBLK, EB)
    dst3 = edge_index[1].reshape(NW, NBLK, EB)
    q0, kv0 = _proj(x, Wq0, bq0, Wk0, bk0, Wv0, bv0)
    e0, e1 = _edge_embed(edge_attr, We0, be0, We1, be1)
    p0 = _sc_attention(q0, kv0, e0, src3, dst3)
    q1, kv1 = _mid(p0, Wq1, bq1, Wk1, bk1, Wv1, bv1)
    p1 = _sc_attention(q1, kv1, e1, src3, dst3)
    return _final(p1)


# speedup vs baseline: 11.5131x; 1.0086x over previous
import functools

import jax
import jax.numpy as jnp
from jax import lax
from jax.experimental import pallas as pl
from jax.experimental.pallas import tpu as pltpu
from jax.experimental.pallas import tpu_sc as plsc

N_NODES = 10000
N_EDGES = 320000
D_IN = 128
D_EDGE = 16
D = 64

NC = 2
NS = 16
NW = NC * NS
EPW = N_EDGES // NW
EB = 80
NBLK = EPW // EB
ACC_W = 80
RCH = 80
NCHT = N_NODES // RCH
NCH_LOOP = -(-NCHT // NS)


def _proj_body(x_ref, wq_ref, bq_ref, wk_ref, bk_ref, wv_ref, bv_ref,
               q_ref, kv_ref):
    xb = x_ref[...]
    q_ref[...] = (jnp.dot(xb, wq_ref[...], preferred_element_type=jnp.float32)
                  + bq_ref[...]) * 0.125
    kb = jnp.dot(xb, wk_ref[...], preferred_element_type=jnp.float32) + bk_ref[...]
    vb = jnp.dot(xb, wv_ref[...], preferred_element_type=jnp.float32) + bv_ref[...]
    kv_ref[...] = jnp.concatenate([kb, vb], axis=1)


def _proj(x, wq, bq, wk, bk, wv, bv):
    n, din = x.shape
    rb = 2000
    w_spec = pl.BlockSpec((din, D), lambda i: (0, 0))
    b_spec = pl.BlockSpec((1, D), lambda i: (0, 0))
    return pl.pallas_call(
        _proj_body,
        grid=(n // rb,),
        in_specs=[pl.BlockSpec((rb, din), lambda i: (i, 0)),
                  w_spec, b_spec, w_spec, b_spec, w_spec, b_spec],
        out_specs=[pl.BlockSpec((rb, D), lambda i: (i, 0)),
                   pl.BlockSpec((rb, 2 * D), lambda i: (i, 0))],
        out_shape=[jax.ShapeDtypeStruct((n, D), jnp.float32),
                   jax.ShapeDtypeStruct((n, 2 * D), jnp.float32)],
    )(x, wq, bq.reshape(1, D), wk, bk.reshape(1, D), wv, bv.reshape(1, D))


def _edge_body(a_ref, w0_ref, b0_ref, w1_ref, b1_ref, e0_ref, e1_ref):
    ab = a_ref[...]
    e0_ref[...] = jnp.dot(ab, w0_ref[...], preferred_element_type=jnp.float32) + b0_ref[...]
    e1_ref[...] = jnp.dot(ab, w1_ref[...], preferred_element_type=jnp.float32) + b1_ref[...]


def _edge_embed(edge_attr, w0, b0, w1, b1):
    ea2 = edge_attr.reshape(N_EDGES // 2, 2 * D_EDGE)
    z = jnp.zeros((D_EDGE, D), jnp.float32)
    w20 = jnp.concatenate(
        [jnp.concatenate([w0, z], axis=1), jnp.concatenate([z, w0], axis=1)],
        axis=0)
    w21 = jnp.concatenate(
        [jnp.concatenate([w1, z], axis=1), jnp.concatenate([z, w1], axis=1)],
        axis=0)
    b20 = jnp.concatenate([b0, b0]).reshape(1, 2 * D)
    b21 = jnp.concatenate([b1, b1]).reshape(1, 2 * D)
    rb = 4000
    w_spec = pl.BlockSpec((2 * D_EDGE, 2 * D), lambda i: (0, 0))
    b_spec = pl.BlockSpec((1, 2 * D), lambda i: (0, 0))
    return pl.pallas_call(
        _edge_body,
        grid=(N_EDGES // 2 // rb,),
        in_specs=[pl.BlockSpec((rb, 2 * D_EDGE), lambda i: (i, 0)),
                  w_spec, b_spec, w_spec, b_spec],
        out_specs=[pl.BlockSpec((rb, 2 * D), lambda i: (i, 0))] * 2,
        out_shape=[jax.ShapeDtypeStruct((N_EDGES // 2, 2 * D), jnp.float32)] * 2,
    )(ea2, w20, b20, w21, b21)


def _mid_body(p0_ref, p1_ref, wq_ref, bq_ref, wk_ref, bk_ref, wv_ref, bv_ref,
              q_ref, kv_ref):
    ps = p0_ref[...] + p1_ref[...]
    num = ps[:, :D]
    den = ps[:, D:D + 1]
    h = jnp.maximum(num / (den + 1e-16), 0.0)
    q_ref[...] = (jnp.dot(h, wq_ref[...], preferred_element_type=jnp.float32)
                  + bq_ref[...]) * 0.125
    kb = jnp.dot(h, wk_ref[...], preferred_element_type=jnp.float32) + bk_ref[...]
    vb = jnp.dot(h, wv_ref[...], preferred_element_type=jnp.float32) + bv_ref[...]
    kv_ref[...] = jnp.concatenate([kb, vb], axis=1)


def _mid(p, wq, bq, wk, bk, wv, bv):
    rb = 2000
    nb = N_NODES // rb
    w_spec = pl.BlockSpec((D, D), lambda i: (0, 0))
    b_spec = pl.BlockSpec((1, D), lambda i: (0, 0))
    return pl.pallas_call(
        _mid_body,
        grid=(nb,),
        in_specs=[pl.BlockSpec((rb, ACC_W), lambda i: (i, 0)),
                  pl.BlockSpec((rb, ACC_W), lambda i: (i + nb, 0)),
                  w_spec, b_spec, w_spec, b_spec, w_spec, b_spec],
        out_specs=[pl.BlockSpec((rb, D), lambda i: (i, 0)),
                   pl.BlockSpec((rb, 2 * D), lambda i: (i, 0))],
        out_shape=[jax.ShapeDtypeStruct((N_NODES, D), jnp.float32),
                   jax.ShapeDtypeStruct((N_NODES, 2 * D), jnp.float32)],
    )(p, p, wq, bq.reshape(1, D), wk, bk.reshape(1, D), wv, bv.reshape(1, D))


def _final_body(p0_ref, p1_ref, o_ref):
    ps = p0_ref[...] + p1_ref[...]
    o_ref[...] = ps[:, :D] / (ps[:, D:D + 1] + 1e-16)


def _final(p):
    rb = 2000
    nb = N_NODES // rb
    return pl.pallas_call(
        _final_body,
        grid=(nb,),
        in_specs=[pl.BlockSpec((rb, ACC_W), lambda i: (i, 0)),
                  pl.BlockSpec((rb, ACC_W), lambda i: (i + nb, 0))],
        out_specs=pl.BlockSpec((rb, D), lambda i: (i, 0)),
        out_shape=jax.ShapeDtypeStruct((N_NODES, D), jnp.float32),
    )(p, p)


_GATHER_DNUMS = lax.GatherDimensionNumbers(
    offset_dims=(), collapsed_slice_dims=(0,), start_index_map=(0,))


def _lane_shuffle(vv, idx):
    return lax.gather(vv, idx[:, None], _GATHER_DNUMS, (1,),
                      mode=lax.GatherScatterMode.PROMISE_IN_BOUNDS)

def _sc_attention(q, kv, e2, src3, dst3):
    mesh = plsc.VectorSubcoreMesh(core_axis_name="c", subcore_axis_name="s")

    @functools.partial(
        pl.kernel,
        out_type=jax.ShapeDtypeStruct((NC * N_NODES, ACC_W), jnp.float32),
        mesh=mesh,
        scratch_types=[
            pltpu.VMEM((NBLK, EB), jnp.int32),
            pltpu.VMEM((NBLK, EB), jnp.int32),
            pltpu.VMEM((EB, D), jnp.float32),
            pltpu.VMEM((EB, 2 * D), jnp.float32),
            pltpu.VMEM((EB // 2, 2 * D), jnp.float32),
            pltpu.VMEM((EB, D), jnp.float32),
            pltpu.VMEM((EB, 2 * D), jnp.float32),
            pltpu.VMEM((EB // 2, 2 * D), jnp.float32),
            pltpu.VMEM((EB, ACC_W), jnp.float32),
            pltpu.VMEM((EB, ACC_W), jnp.float32),
            pltpu.VMEM((RCH, ACC_W), jnp.float32),
            pltpu.VMEM_SHARED((N_NODES, ACC_W), jnp.float32),
            pltpu.SemaphoreType.DMA,
            pltpu.SemaphoreType.DMA,
            pltpu.SemaphoreType.DMA,
            pltpu.SemaphoreType.DMA,
            pltpu.SemaphoreType.DMA,
            pltpu.SemaphoreType.DMA,
            pltpu.SemaphoreType.DMA,
            pltpu.SemaphoreType.DMA,
        ],
        compiler_params=pltpu.CompilerParams(use_tc_tiling_on_sc=False),
    )
    def sc_kernel(q_hbm, kv_hbm, e_hbm, src_hbm, dst_hbm, out_hbm,
                  src2d, dst2d, qg0, kvg0, eg0, qg1, kvg1, eg1,
                  sbuf0, sbuf1, zbuf, acc,
                  sq0, skv0, se0, sq1, skv1, se1, ssc0, ssc1):
        c = lax.axis_index("c")
        s = lax.axis_index("s")
        wid = c * NS + s

        @pl.loop(0, RCH)
        def _zero_rows(i):
            for j in range(ACC_W // 16):
                zbuf[i, pl.ds(j * 16, 16)] = jnp.zeros((16,), jnp.float32)

        @pl.loop(0, NCH_LOOP)
        def _zero_acc(jc):
            m = s + jc * NS
            @pl.when(m < NCHT)
            def _():
                pltpu.sync_copy(zbuf, acc.at[pl.ds(m * RCH, RCH)])

        pltpu.sync_copy(src_hbm.at[wid], src2d)
        pltpu.sync_copy(dst_hbm.at[wid], dst2d)

        plsc.subcore_barrier()

        lane = lax.iota(jnp.int32, 16)
        onehot0 = jnp.where(lane == 0, 1.0, 0.0).astype(jnp.float32)
        ebase0 = wid * (EPW // 2)

        slots = ((qg0, kvg0, eg0, sq0, skv0, se0, sbuf0, ssc0),
                 (qg1, kvg1, eg1, sq1, skv1, se1, sbuf1, ssc1))

        def issue(b, slot):
            qg, kvg, eg, sq, skv, se, _, _ = slots[slot]
            pltpu.async_copy(q_hbm.at[dst2d.at[b]], qg, sq)
            pltpu.async_copy(kv_hbm.at[src2d.at[b]], kvg, skv)
            pltpu.async_copy(
                e_hbm.at[pl.ds(ebase0 + b * (EB // 2), EB // 2)], eg, se)

        def wait(slot):
            qg, kvg, eg, sq, skv, se, _, _ = slots[slot]
            pltpu.make_async_copy(q_hbm.at[dst2d.at[0]], qg, sq).wait()
            pltpu.make_async_copy(kv_hbm.at[src2d.at[0]], kvg, skv).wait()
            pltpu.make_async_copy(
                e_hbm.at[pl.ds(0, EB // 2)], eg, se).wait()

        def compute_and_scatter(b, slot):
            qg, kvg, eg, _, _, _, sbuf, ssc = slots[slot]

            @pl.when(b >= 2)
            def _():
                pltpu.make_async_copy(sbuf, acc.at[dst2d.at[0]], ssc).wait()

            @pl.loop(0, EB // 2, unroll=2)
            def _edge_pairs(j):
                for half in range(2):
                    i = 2 * j + half
                    eoff = half * D
                    ev = [eg[j, pl.ds(eoff + 16 * t, 16)] for t in range(4)]
                    prod = (qg[i, pl.ds(0, 16)] * (kvg[i, pl.ds(0, 16)] + ev[0])
                            + qg[i, pl.ds(16, 16)] * (kvg[i, pl.ds(16, 16)] + ev[1])
                            + qg[i, pl.ds(32, 16)] * (kvg[i, pl.ds(32, 16)] + ev[2])
                            + qg[i, pl.ds(48, 16)] * (kvg[i, pl.ds(48, 16)] + ev[3]))
                    for sh in (8, 4, 2, 1):
                        idx = lane ^ sh
                        prod = prod + _lane_shuffle(prod, idx)
                    ew = jnp.exp(prod)
                    for t in range(4):
                        sbuf[i, pl.ds(16 * t, 16)] = ew * (
                            kvg[i, pl.ds(D + 16 * t, 16)] + ev[t])
                    sbuf[i, pl.ds(D, 16)] = ew * onehot0

            pltpu.async_copy(sbuf, acc.at[dst2d.at[b]], ssc, add=True)

        issue(0, 0)

        @pl.loop(0, (NBLK + 1) // 2)
        def _pairs(p):
            b0 = 2 * p
            b1 = b0 + 1

            @pl.when(b1 < NBLK)
            def _():
                issue(b1, 1)

            wait(0)
            compute_and_scatter(b0, 0)

            @pl.when(b1 < NBLK)
            def _():
                @pl.when(b1 + 1 < NBLK)
                def _():
                    issue(b1 + 1, 0)

                wait(1)
                compute_and_scatter(b1, 1)

        pltpu.make_async_copy(sbuf0, acc.at[dst2d.at[0]], ssc0).wait()
        pltpu.make_async_copy(sbuf1, acc.at[dst2d.at[0]], ssc1).wait()

        plsc.subcore_barrier()

        @pl.loop(0, NCH_LOOP)
        def _copy_out(jc):
            m = s + jc * NS
            @pl.when(m < NCHT)
            def _():
                off = m * RCH
                pltpu.sync_copy(acc.at[pl.ds(off, RCH)], zbuf)
                pltpu.sync_copy(zbuf, out_hbm.at[pl.ds(c * N_NODES + off, RCH)])

    return sc_kernel(q, kv, e2, src3, dst3)


def kernel(x, edge_index, edge_attr, Wq0, bq0, Wk0, bk0, Wv0, bv0, We0, be0,
           Wq1, bq1, Wk1, bk1, Wv1, bv1, We1, be1):
    src3 = edge_index[0].reshape(NW, NBLK, EB)
    dst3 = edge_index[1].reshape(NW, NBLK, EB)
    q0, kv0 = _proj(x, Wq0, bq0, Wk0, bk0, Wv0, bv0)
    e0, e1 = _edge_embed(edge_attr, We0, be0, We1, be1)
    p0 = _sc_attention(q0, kv0, e0, src3, dst3)
    q1, kv1 = _mid(p0, Wq1, bq1, Wk1, bk1, Wv1, bv1)
    p1 = _sc_attention(q1, kv1, e1, src3, dst3)
    return _final(p1)
